# baseline (speedup 1.0000x reference)
import functools

import jax
import jax.numpy as jnp
from jax.experimental import pallas as pl

_BIG = 3e37
_BIGX = 1e15


def _dtw_wavefront(krot_ref, xr_ref, out_ref, *, m, n):
    big = jnp.float32(_BIG)
    lane = jax.lax.broadcasted_iota(jnp.int32, (1, m), 1)
    ndiag = m + n - 1
    nblocks = ndiag // m
    rem = ndiag - nblocks * m

    def steps(U, RU, r0, r1, t_lo, t_hi):
        for t in range(t_lo, t_hi):
            kr = krot_ref[pl.ds(t, 1), :]
            w = jnp.where(lane < t, r1, r0) if t else r0
            c = (kr - w) ** 2
            RU1 = jnp.roll(U, 1, axis=1)
            ud = jnp.where(lane == (t - 1) % m, big, jnp.minimum(U, RU))
            U = c + jnp.minimum(RU1, ud)
            RU = RU1
        return U, RU

    r0 = xr_ref[pl.ds(0, 1), :]
    r1 = xr_ref[pl.ds(1, 1), :]
    c0 = (krot_ref[pl.ds(0, 1), :] - r0) ** 2
    U = c0 + jnp.where(lane == m - 1, jnp.float32(0), big)
    RU = jnp.full((1, m), big, jnp.float32)
    U, RU = steps(U, RU, r0, r1, 1, m)

    def block(b, carry):
        U, RU = carry
        r0 = xr_ref[pl.ds(b, 1), :]
        r1 = xr_ref[pl.ds(b + 1, 1), :]
        return steps(U, RU, r0, r1, 0, m)

    U, RU = jax.lax.fori_loop(1, nblocks, block, (U, RU))
    if rem:
        r0 = xr_ref[pl.ds(nblocks, 1), :]
        r1 = xr_ref[pl.ds(nblocks + 1, 1), :]
        U, RU = steps(U, RU, r0, r1, 0, rem)
    p_fin = (ndiag - 1) % m
    out_ref[...] = U[:, p_fin : p_fin + 1]


def kernel(x, kernel):
    m = kernel.shape[0]
    n = x.shape[0]
    krev = kernel[::-1].astype(jnp.float32)
    k2 = jnp.concatenate([krev, krev])
    krot = jnp.stack([jax.lax.slice(k2, (m - r,), (2 * m - r,)) for r in range(m)])
    nrows = (n + 3 * m - 3) // m + 1
    lpad = jnp.full((m - 1,), _BIGX, jnp.float32)
    rpad = jnp.full((nrows * m - (m - 1) - n,), _BIGX, jnp.float32)
    xr = jnp.concatenate([lpad, x.astype(jnp.float32), rpad]).reshape(nrows, m)
    out = pl.pallas_call(
        functools.partial(_dtw_wavefront, m=m, n=n),
        out_shape=jax.ShapeDtypeStruct((1, 1), jnp.float32),
    )(krot, xr)
    return out[0, 0]


# speedup vs baseline: 2.5590x; 2.5590x over previous
import functools

import jax
import jax.numpy as jnp
import numpy as np
from jax.experimental import pallas as pl

_BIG = 3e37
_BIGX = 1e15
_GK = 16


def _dtw_wavefront(krot_ref, xr_ref, out_ref, *, m, n):
    big = jnp.float32(_BIG)
    lane = jax.lax.broadcasted_iota(jnp.int32, (1, m), 1)

    def mask_eq(c):
        return lane == c

    def mask_lt(c):
        return lane < c

    def roll(v, s):
        return jnp.roll(v, s, axis=1)

    def cost_row(t, r0, r1):
        kr = krot_ref[pl.ds(t, 1), :]
        w = jnp.where(mask_lt(t), r1, r0) if t else r0
        return (kr - w) ** 2

    def group(A, B, ts, costs):
        gk = len(ts)
        RA, RB = [A], [B]
        for s in range(1, gk + 1):
            RA.append(roll(A, s))
            RB.append(roll(B, s))
        U = {}
        for idx, t in enumerate(ts):
            j = idx + 1
            c0 = costs[idx]
            for s in range(gk - j + 1):
                c = roll(c0, s) if s else c0
                uL_s1 = RA[s + 1] if j == 1 else U[(j - 1, s + 1)]
                uL_s = RA[s] if j == 1 else U[(j - 1, s)]
                if j == 1:
                    uLL = RB[s + 1]
                elif j == 2:
                    uLL = RA[s + 1]
                else:
                    uLL = U[(j - 2, s + 1)]
                mask = mask_eq((t - 1 + s) % m)
                U[(j, s)] = c + jnp.minimum(
                    uL_s1, jnp.where(mask, big, jnp.minimum(uL_s, uLL))
                )
        return U[(gk, 0)], U[(gk - 1, 0)]

    def chunks(t_list):
        return [t_list[i : i + _GK] for i in range(0, len(t_list), _GK)]

    r0 = xr_ref[pl.ds(0, 1), :]
    c0 = (krot_ref[pl.ds(0, 1), :] - r0) ** 2
    A = c0 + jnp.where(mask_eq(m - 1), jnp.float32(0), big)
    B = jnp.full((1, m), big, jnp.float32)

    ndiag = m + n - 1
    nfull = (ndiag - 1) // m
    tail = ndiag - 1 - nfull * m

    def block(b, carry):
        A, B = carry
        r0 = xr_ref[pl.ds(b, 1), :]
        r1 = xr_ref[pl.ds(b + 1, 1), :]
        for ts in chunks(list(range(1, m)) + [0]):
            costs = [cost_row(t, r0, r1) if t else cost_row(0, r1, r1) for t in ts]
            A, B = group(A, B, ts, costs)
        return A, B

    A, B = jax.lax.fori_loop(0, nfull, block, (A, B))
    if tail:
        r0 = xr_ref[pl.ds(nfull, 1), :]
        r1 = xr_ref[pl.ds(nfull + 1, 1), :]
        for ts in chunks(list(range(1, tail + 1))):
            costs = [cost_row(t, r0, r1) for t in ts]
            A, B = group(A, B, ts, costs)
    p_fin = (ndiag - 1) % m
    out_ref[...] = A[:, p_fin : p_fin + 1]


def kernel(x, kernel):
    m = kernel.shape[0]
    n = x.shape[0]
    krev = kernel[::-1].astype(jnp.float32)
    k2 = jnp.concatenate([krev, krev])
    krot = jnp.stack([jax.lax.slice(k2, (m - r,), (2 * m - r,)) for r in range(m)])
    nrows = (n + 3 * m - 3) // m + 1
    lpad = jnp.full((m - 1,), _BIGX, jnp.float32)
    rpad = jnp.full((nrows * m - (m - 1) - n,), _BIGX, jnp.float32)
    xr = jnp.concatenate([lpad, x.astype(jnp.float32), rpad]).reshape(nrows, m)
    out = pl.pallas_call(
        functools.partial(_dtw_wavefront, m=m, n=n),
        out_shape=jax.ShapeDtypeStruct((1, 1), jnp.float32),
    )(krot, xr)
    return out[0, 0]


# speedup vs baseline: 3.9993x; 1.5628x over previous
import functools

import jax
import jax.numpy as jnp
from jax.experimental import pallas as pl

_BIG = 3e37
_BIGX = 1e15
_GK = 8


def _dtw_wavefront(krot_ref, xs_ref, out_ref, *, m, n):
    big = jnp.float32(_BIG)
    lane = jax.lax.broadcasted_iota(jnp.int32, (1, m), 1)

    def mask_eq(c):
        return lane == c

    def roll(v, s):
        return jnp.roll(v, s, axis=1)

    def cost_sv(t, s, rows):
        kr = krot_ref[pl.ds((t + s) % m, 1), :]
        ra, rb, rc = rows[s]
        if t:
            thr = s + t
            w = rb if thr >= m else (jnp.where(lane < thr, rb, ra) if thr else ra)
        else:
            w = jnp.where(lane < s, rc, rb) if s else rb
        return (kr - w) ** 2

    def group(A, B, ts, rows):
        gk = len(ts)
        RA, RB = [A], [B]
        for s in range(1, gk + 1):
            RA.append(roll(A, s))
            RB.append(roll(B, s))
        U = {}
        for idx, t in enumerate(ts):
            j = idx + 1
            for s in range(gk - j + 1):
                c = cost_sv(t, s, rows)
                uL_s1 = RA[s + 1] if j == 1 else U[(j - 1, s + 1)]
                uL_s = RA[s] if j == 1 else U[(j - 1, s)]
                if j == 1:
                    uLL = RB[s + 1]
                elif j == 2:
                    uLL = RA[s + 1]
                else:
                    uLL = U[(j - 2, s + 1)]
                mask = mask_eq((t - 1 + s) % m)
                U[(j, s)] = c + jnp.minimum(
                    uL_s1, jnp.where(mask, big, jnp.minimum(uL_s, uLL))
                )
        return U[(gk, 0)], U[(gk - 1, 0)]

    def chunks(t_list):
        return [t_list[i : i + _GK] for i in range(0, len(t_list), _GK)]

    def load_rows(b):
        return [
            (
                xs_ref[s, pl.ds(b, 1), :],
                xs_ref[s, pl.ds(b + 1, 1), :],
                xs_ref[s, pl.ds(b + 2, 1), :],
            )
            for s in range(_GK + 1)
        ]

    r0 = xs_ref[0, pl.ds(0, 1), :]
    c0 = (krot_ref[pl.ds(0, 1), :] - r0) ** 2
    A = c0 + jnp.where(mask_eq(m - 1), jnp.float32(0), big)
    B = jnp.full((1, m), big, jnp.float32)

    ndiag = m + n - 1
    nfull = (ndiag - 1) // m
    tail = ndiag - 1 - nfull * m

    def block(b, carry):
        A, B = carry
        rows = load_rows(b)
        for ts in chunks(list(range(1, m)) + [0]):
            A, B = group(A, B, ts, rows)
        return A, B

    A, B = jax.lax.fori_loop(0, nfull, block, (A, B))
    if tail:
        rows = load_rows(nfull)
        for ts in chunks(list(range(1, tail + 1))):
            A, B = group(A, B, ts, rows)
    p_fin = (ndiag - 1) % m
    out_ref[...] = A[:, p_fin : p_fin + 1]


def kernel(x, kernel):
    m = kernel.shape[0]
    n = x.shape[0]
    krev = kernel[::-1].astype(jnp.float32)
    k2 = jnp.concatenate([krev, krev])
    krot = jnp.stack([jax.lax.slice(k2, (m - r,), (2 * m - r,)) for r in range(m)])
    nrows = (n + 3 * m - 3) // m + 1
    lpad = jnp.full((m - 1 + _GK,), _BIGX, jnp.float32)
    rpad = jnp.full((nrows * m - (m - 1) - n,), _BIGX, jnp.float32)
    xpad2 = jnp.concatenate([lpad, x.astype(jnp.float32), rpad])
    xs = jnp.stack(
        [
            jax.lax.slice(xpad2, (_GK - s,), (_GK - s + nrows * m,)).reshape(nrows, m)
            for s in range(_GK + 1)
        ]
    )
    out = pl.pallas_call(
        functools.partial(_dtw_wavefront, m=m, n=n),
        out_shape=jax.ShapeDtypeStruct((1, 1), jnp.float32),
    )(krot, xs)
    return out[0, 0]


# speedup vs baseline: 4.2620x; 1.0657x over previous
import functools

import jax
import jax.numpy as jnp
from jax.experimental import pallas as pl

_BIG = 3e37
_BIGX = 1e15
_GK = 12


def _dtw_wavefront(krot_ref, xs_ref, out_ref, *, m, n):
    big = jnp.float32(_BIG)
    lane = jax.lax.broadcasted_iota(jnp.int32, (1, m), 1)

    def mask_eq(c):
        return lane == c

    def roll(v, s):
        return jnp.roll(v, s, axis=1)

    def cost_sv(t, s, rows):
        kr = krot_ref[pl.ds((t + s) % m, 1), :]
        ra, rb, rc = rows[s]
        if t:
            thr = s + t
            w = rb if thr >= m else (jnp.where(lane < thr, rb, ra) if thr else ra)
        else:
            w = jnp.where(lane < s, rc, rb) if s else rb
        return (kr - w) ** 2

    def group(A, B, ts, rows):
        gk = len(ts)
        RA, RB = [A], [B]
        for s in range(1, gk + 1):
            RA.append(roll(A, s))
            RB.append(roll(B, s))
        U = {}
        for idx, t in enumerate(ts):
            j = idx + 1
            for s in range(gk - j + 1):
                c = cost_sv(t, s, rows)
                uL_s1 = RA[s + 1] if j == 1 else U[(j - 1, s + 1)]
                uL_s = RA[s] if j == 1 else U[(j - 1, s)]
                if j == 1:
                    uLL = RB[s + 1]
                elif j == 2:
                    uLL = RA[s + 1]
                else:
                    uLL = U[(j - 2, s + 1)]
                mask = mask_eq((t - 1 + s) % m)
                U[(j, s)] = c + jnp.minimum(
                    uL_s1, jnp.where(mask, big, jnp.minimum(uL_s, uLL))
                )
        return U[(gk, 0)], U[(gk - 1, 0)]

    def chunks(t_list):
        return [t_list[i : i + _GK] for i in range(0, len(t_list), _GK)]

    def load_rows(b):
        return [
            (
                xs_ref[s, pl.ds(b, 1), :],
                xs_ref[s, pl.ds(b + 1, 1), :],
                xs_ref[s, pl.ds(b + 2, 1), :],
            )
            for s in range(_GK + 1)
        ]

    r0 = xs_ref[0, pl.ds(0, 1), :]
    c0 = (krot_ref[pl.ds(0, 1), :] - r0) ** 2
    A = c0 + jnp.where(mask_eq(m - 1), jnp.float32(0), big)
    B = jnp.full((1, m), big, jnp.float32)

    ndiag = m + n - 1
    nfull = (ndiag - 1) // m
    tail = ndiag - 1 - nfull * m

    def block(b, carry):
        A, B = carry
        rows = load_rows(b)
        for ts in chunks(list(range(1, m)) + [0]):
            A, B = group(A, B, ts, rows)
        return A, B

    A, B = jax.lax.fori_loop(0, nfull, block, (A, B))
    if tail:
        rows = load_rows(nfull)
        for ts in chunks(list(range(1, tail + 1))):
            A, B = group(A, B, ts, rows)
    p_fin = (ndiag - 1) % m
    out_ref[...] = A[:, p_fin : p_fin + 1]


def kernel(x, kernel):
    m = kernel.shape[0]
    n = x.shape[0]
    krev = kernel[::-1].astype(jnp.float32)
    k2 = jnp.concatenate([krev, krev])
    krot = jnp.stack([jax.lax.slice(k2, (m - r,), (2 * m - r,)) for r in range(m)])
    nrows = (n + 3 * m - 3) // m + 1
    lpad = jnp.full((m - 1 + _GK,), _BIGX, jnp.float32)
    rpad = jnp.full((nrows * m - (m - 1) - n,), _BIGX, jnp.float32)
    xpad2 = jnp.concatenate([lpad, x.astype(jnp.float32), rpad])
    xs = jnp.stack(
        [
            jax.lax.slice(xpad2, (_GK - s,), (_GK - s + nrows * m,)).reshape(nrows, m)
            for s in range(_GK + 1)
        ]
    )
    out = pl.pallas_call(
        functools.partial(_dtw_wavefront, m=m, n=n),
        out_shape=jax.ShapeDtypeStruct((1, 1), jnp.float32),
    )(krot, xs)
    return out[0, 0]


# speedup vs baseline: 4.4238x; 1.0380x over previous
import functools

import jax
import jax.numpy as jnp
from jax.experimental import pallas as pl

_BIG = 3e37
_BIGX = 1e15
_GK = 16


def _dtw_wavefront(krot_ref, xs_ref, out_ref, *, m, n):
    big = jnp.float32(_BIG)
    lane = jax.lax.broadcasted_iota(jnp.int32, (1, m), 1)

    def mask_eq(c):
        return lane == c

    def roll(v, s):
        return jnp.roll(v, s, axis=1)

    def cost_sv(t, s, rows):
        kr = krot_ref[pl.ds((t + s) % m, 1), :]
        ra, rb, rc = rows[s]
        if t:
            thr = s + t
            w = rb if thr >= m else (jnp.where(lane < thr, rb, ra) if thr else ra)
        else:
            w = jnp.where(lane < s, rc, rb) if s else rb
        return (kr - w) ** 2

    def group(A, B, ts, rows):
        gk = len(ts)
        RA, RB = [A], [B]
        for s in range(1, gk + 1):
            RA.append(roll(A, s))
            RB.append(roll(B, s))
        U = {}
        for idx, t in enumerate(ts):
            j = idx + 1
            for s in range(gk - j + 1):
                c = cost_sv(t, s, rows)
                uL_s1 = RA[s + 1] if j == 1 else U[(j - 1, s + 1)]
                uL_s = RA[s] if j == 1 else U[(j - 1, s)]
                if j == 1:
                    uLL = RB[s + 1]
                elif j == 2:
                    uLL = RA[s + 1]
                else:
                    uLL = U[(j - 2, s + 1)]
                mask = mask_eq((t - 1 + s) % m)
                U[(j, s)] = c + jnp.minimum(
                    uL_s1, jnp.where(mask, big, jnp.minimum(uL_s, uLL))
                )
        return U[(gk, 0)], U[(gk - 1, 0)]

    def chunks(t_list):
        return [t_list[i : i + _GK] for i in range(0, len(t_list), _GK)]

    def load_rows(b):
        return [
            (
                xs_ref[s, pl.ds(b, 1), :],
                xs_ref[s, pl.ds(b + 1, 1), :],
                xs_ref[s, pl.ds(b + 2, 1), :],
            )
            for s in range(_GK + 1)
        ]

    r0 = xs_ref[0, pl.ds(0, 1), :]
    c0 = (krot_ref[pl.ds(0, 1), :] - r0) ** 2
    A = c0 + jnp.where(mask_eq(m - 1), jnp.float32(0), big)
    B = jnp.full((1, m), big, jnp.float32)

    ndiag = m + n - 1
    nfull = (ndiag - 1) // m
    tail = ndiag - 1 - nfull * m

    def block(b, carry):
        A, B = carry
        rows = load_rows(b)
        for ts in chunks(list(range(1, m)) + [0]):
            A, B = group(A, B, ts, rows)
        return A, B

    A, B = jax.lax.fori_loop(0, nfull, block, (A, B))
    if tail:
        rows = load_rows(nfull)
        for ts in chunks(list(range(1, tail + 1))):
            A, B = group(A, B, ts, rows)
    p_fin = (ndiag - 1) % m
    out_ref[...] = A[:, p_fin : p_fin + 1]


def kernel(x, kernel):
    m = kernel.shape[0]
    n = x.shape[0]
    krev = kernel[::-1].astype(jnp.float32)
    k2 = jnp.concatenate([krev, krev])
    krot = jnp.stack([jax.lax.slice(k2, (m - r,), (2 * m - r,)) for r in range(m)])
    nrows = (n + 3 * m - 3) // m + 1
    lpad = jnp.full((m - 1 + _GK,), _BIGX, jnp.float32)
    rpad = jnp.full((nrows * m - (m - 1) - n,), _BIGX, jnp.float32)
    xpad2 = jnp.concatenate([lpad, x.astype(jnp.float32), rpad])
    xs = jnp.stack(
        [
            jax.lax.slice(xpad2, (_GK - s,), (_GK - s + nrows * m,)).reshape(nrows, m)
            for s in range(_GK + 1)
        ]
    )
    out = pl.pallas_call(
        functools.partial(_dtw_wavefront, m=m, n=n),
        out_shape=jax.ShapeDtypeStruct((1, 1), jnp.float32),
    )(krot, xs)
    return out[0, 0]
